# baseline (speedup 1.0000x reference)
import functools

import jax
import jax.numpy as jnp
from jax import lax
from jax.experimental import pallas as pl
from jax.experimental.pallas import tpu as pltpu
from jax.experimental.pallas import tpu_sc as plsc

_UNITS = 128
_N_ROWS = 160000
_TOTAL = _N_ROWS * _UNITS

_SC_ROWS = 40000
_TC_ROWS = _N_ROWS - _SC_ROWS
_SC_BASE = _TC_ROWS * _UNITS
_SC_ELEMS = _SC_ROWS * _UNITS

_ROWS_PER_BLOCK = 2000

_NW = 32
_PER_W = _SC_ELEMS // _NW
_CHUNK = 8000
_N_CHUNK = _PER_W // _CHUNK
_UNROLL = 4

_KS = (0, 42, 0x1BD11BDA ^ 42)
_ROT = ((13, 15, 26, 6), (17, 29, 16, 24))
_INJ = ((1, 2, 0), (2, 0, 1), (0, 1, 0), (1, 2, 1), (2, 0, 0))


def _rotl(x, r):
    return (x << r) | lax.shift_right_logical(x, 32 - r)


def _rounds(x0, x1, rots):
    for r in rots:
        x0 = x0 + x1
        x1 = _rotl(x1, r) ^ x0
    return x0, x1


def _threefry_bits(ctr):
    x0 = ctr
    x1 = _rotl(ctr, _ROT[0][0]) ^ ctr
    x0, x1 = _rounds(x0, x1, _ROT[0][1:])
    for i, (a, b, grp) in enumerate(_INJ, start=1):
        x0 = x0 + jnp.int32(_KS[a])
        x1 = x1 + jnp.int32(_KS[b] + i)
        if i < 5:
            x0, x1 = _rounds(x0, x1, _ROT[(grp + 1) % 2])
    return x0 ^ x1


def _tc_body(v_ref, o_ref):
    pid = pl.program_id(0)
    shape = v_ref.shape
    base = pid * (shape[0] * shape[1])
    row = lax.broadcasted_iota(jnp.int32, shape, 0)
    col = lax.broadcasted_iota(jnp.int32, shape, 1)
    ctr = (base + _KS[1]) + ((row << 7) | col)
    bits = _threefry_bits(ctr)
    o_ref[...] = jnp.where(bits >= 0, v_ref[...] * 2.0, 0.0)


def _tc_call(values):
    vals = values.reshape(_N_ROWS, _UNITS)
    return pl.pallas_call(
        _tc_body,
        grid=(_TC_ROWS // _ROWS_PER_BLOCK,),
        in_specs=[pl.BlockSpec((_ROWS_PER_BLOCK, _UNITS), lambda i: (i, 0))],
        out_specs=pl.BlockSpec((_ROWS_PER_BLOCK, _UNITS), lambda i: (i, 0)),
        out_shape=jax.ShapeDtypeStruct((_TC_ROWS, _UNITS), jnp.float32),
        compiler_params=pltpu.CompilerParams(
            dimension_semantics=("parallel",)),
    )(vals)


def _sc_kernel_body(v_hbm, out_hbm, vin, vout):
    wid = lax.axis_index("s") * 2 + lax.axis_index("c")
    wbase = wid * _PER_W

    def chunk_body(c, _):
        off = wbase + c * _CHUNK
        pltpu.sync_copy(v_hbm.at[pl.ds(_SC_BASE + off, _CHUNK)], vin)

        def vec_body(j, _):
            s = off + j * (16 * _UNROLL)
            iota = lax.iota(jnp.int32, 16)
            for u in range(_UNROLL):
                ctr = iota + jnp.int32(_SC_BASE + _KS[1] + u * 16) + s
                bits = _threefry_bits(ctr)
                sl = pl.ds(j * (16 * _UNROLL) + u * 16, 16)
                vout[sl] = jnp.where(bits >= 0, vin[sl] * 2.0, 0.0)
            return 0

        lax.fori_loop(0, _CHUNK // (16 * _UNROLL), vec_body, 0)
        pltpu.sync_copy(vout, out_hbm.at[pl.ds(off, _CHUNK)])
        return 0

    lax.fori_loop(0, _N_CHUNK, chunk_body, 0)


def _sc_call(values):
    mesh = plsc.VectorSubcoreMesh(core_axis_name="c", subcore_axis_name="s")
    k = functools.partial(
        pl.kernel,
        mesh=mesh,
        out_type=jax.ShapeDtypeStruct((_SC_ELEMS,), jnp.float32),
        scratch_types=[
            pltpu.VMEM((_CHUNK,), jnp.float32),
            pltpu.VMEM((_CHUNK,), jnp.float32),
        ],
    )(_sc_kernel_body)
    return k(values)


def kernel(values, indices):
    del indices
    tc_out = _tc_call(values)
    sc_out = _sc_call(values)
    return jnp.concatenate([tc_out.reshape(-1), sc_out])


# speedup vs baseline: 1.1853x; 1.1853x over previous
import functools

import jax
import jax.numpy as jnp
from jax import lax
from jax.experimental import pallas as pl
from jax.experimental.pallas import tpu as pltpu
from jax.experimental.pallas import tpu_sc as plsc

_UNITS = 128
_N_ROWS = 160000
_TOTAL = _N_ROWS * _UNITS

_SC_ROWS = 38000
_TC_ROWS = _N_ROWS - _SC_ROWS
_SC_BASE = _TC_ROWS * _UNITS
_SC_ELEMS = _SC_ROWS * _UNITS

_ROWS_PER_BLOCK = 2000

_NW = 32
_PER_W = _SC_ELEMS // _NW
_CHUNK = 8000
_N_CHUNK = _PER_W // _CHUNK
_UNROLL = 4

_KS = (0, 42, 0x1BD11BDA ^ 42)
_ROT = ((13, 15, 26, 6), (17, 29, 16, 24))
_INJ = ((1, 2, 0), (2, 0, 1), (0, 1, 0), (1, 2, 1), (2, 0, 0))


def _rotl(x, r):
    return (x << r) | lax.shift_right_logical(x, 32 - r)


def _rounds(x0, x1, rots):
    for r in rots:
        x0 = x0 + x1
        x1 = _rotl(x1, r) ^ x0
    return x0, x1


def _threefry_bits(ctr):
    x0 = ctr
    x1 = _rotl(ctr, _ROT[0][0]) ^ ctr
    x0, x1 = _rounds(x0, x1, _ROT[0][1:])
    for i, (a, b, grp) in enumerate(_INJ, start=1):
        x0 = x0 + jnp.int32(_KS[a])
        x1 = x1 + jnp.int32(_KS[b] + i)
        if i < 5:
            x0, x1 = _rounds(x0, x1, _ROT[(grp + 1) % 2])
    return x0 ^ x1


def _tc_body(v_ref, o_ref):
    pid = pl.program_id(0)
    shape = v_ref.shape
    base = pid * (shape[0] * shape[1])
    row = lax.broadcasted_iota(jnp.int32, shape, 0)
    col = lax.broadcasted_iota(jnp.int32, shape, 1)
    ctr = (base + _KS[1]) + ((row << 7) | col)
    bits = _threefry_bits(ctr)
    o_ref[...] = jnp.where(bits >= 0, v_ref[...] * 2.0, 0.0)


def _tc_call(values):
    vals = values.reshape(_N_ROWS, _UNITS)
    return pl.pallas_call(
        _tc_body,
        grid=(_TC_ROWS // _ROWS_PER_BLOCK,),
        in_specs=[pl.BlockSpec((_ROWS_PER_BLOCK, _UNITS), lambda i: (i, 0))],
        out_specs=pl.BlockSpec((_ROWS_PER_BLOCK, _UNITS), lambda i: (i, 0)),
        out_shape=jax.ShapeDtypeStruct((_N_ROWS, _UNITS), jnp.float32),
        compiler_params=pltpu.CompilerParams(
            dimension_semantics=("parallel",)),
    )(vals)


def _sc_kernel_body(v_hbm, out_hbm, vin, vout):
    wid = lax.axis_index("s") * 2 + lax.axis_index("c")
    wbase = wid * _PER_W

    def chunk_body(c, _):
        off = wbase + c * _CHUNK
        pltpu.sync_copy(v_hbm.at[pl.ds(_SC_BASE + off, _CHUNK)], vin)

        def vec_body(j, _):
            s = off + j * (16 * _UNROLL)
            iota = lax.iota(jnp.int32, 16)
            for u in range(_UNROLL):
                ctr = iota + jnp.int32(_SC_BASE + _KS[1] + u * 16) + s
                bits = _threefry_bits(ctr)
                sl = pl.ds(j * (16 * _UNROLL) + u * 16, 16)
                vout[sl] = jnp.where(bits >= 0, vin[sl] * 2.0, 0.0)
            return 0

        lax.fori_loop(0, _CHUNK // (16 * _UNROLL), vec_body, 0)
        pltpu.sync_copy(vout, out_hbm.at[pl.ds(off, _CHUNK)])
        return 0

    lax.fori_loop(0, _N_CHUNK, chunk_body, 0)


def _sc_call(values):
    mesh = plsc.VectorSubcoreMesh(core_axis_name="c", subcore_axis_name="s")
    k = functools.partial(
        pl.kernel,
        mesh=mesh,
        out_type=jax.ShapeDtypeStruct((_SC_ELEMS,), jnp.float32),
        scratch_types=[
            pltpu.VMEM((_CHUNK,), jnp.float32),
            pltpu.VMEM((_CHUNK,), jnp.float32),
        ],
    )(_sc_kernel_body)
    return k(values)


def kernel(values, indices):
    del indices
    tc_out = _tc_call(values)
    sc_out = _sc_call(values)
    return lax.dynamic_update_slice(tc_out.reshape(-1), sc_out, (_SC_BASE,))


# speedup vs baseline: 1.1893x; 1.0034x over previous
import functools

import jax
import jax.numpy as jnp
from jax import lax
from jax.experimental import pallas as pl
from jax.experimental.pallas import tpu as pltpu
from jax.experimental.pallas import tpu_sc as plsc

_UNITS = 128
_N_ROWS = 160000
_TOTAL = _N_ROWS * _UNITS

_SC_ROWS = 38400
_TC_ROWS = _N_ROWS - _SC_ROWS
_SC_BASE = _TC_ROWS * _UNITS
_SC_ELEMS = _SC_ROWS * _UNITS

_ROWS_PER_BLOCK = 1600

_NW = 32
_PER_W = _SC_ELEMS // _NW
_CHUNK = 9600
_N_CHUNK = _PER_W // _CHUNK
_N_PAIR = _N_CHUNK // 2
_UNROLL = 4

_KS = (0, 42, 0x1BD11BDA ^ 42)
_ROT = ((13, 15, 26, 6), (17, 29, 16, 24))
_INJ = ((1, 2, 0), (2, 0, 1), (0, 1, 0), (1, 2, 1), (2, 0, 0))


def _rotl(x, r):
    return (x << r) | lax.shift_right_logical(x, 32 - r)


def _rounds(x0, x1, rots):
    for r in rots:
        x0 = x0 + x1
        x1 = _rotl(x1, r) ^ x0
    return x0, x1


def _threefry_bits(ctr):
    x0 = ctr
    x1 = _rotl(ctr, _ROT[0][0]) ^ ctr
    x0, x1 = _rounds(x0, x1, _ROT[0][1:])
    for i, (a, b, grp) in enumerate(_INJ, start=1):
        x0 = x0 + jnp.int32(_KS[a])
        x1 = x1 + jnp.int32(_KS[b] + i)
        if i < 5:
            x0, x1 = _rounds(x0, x1, _ROT[(grp + 1) % 2])
    return x0 ^ x1


def _tc_body(v_ref, o_ref):
    pid = pl.program_id(0)
    shape = v_ref.shape
    base = pid * (shape[0] * shape[1])
    row = lax.broadcasted_iota(jnp.int32, shape, 0)
    col = lax.broadcasted_iota(jnp.int32, shape, 1)
    ctr = (base + _KS[1]) + ((row << 7) | col)
    bits = _threefry_bits(ctr)
    o_ref[...] = jnp.where(bits >= 0, v_ref[...] * 2.0, 0.0)


def _tc_call(values):
    vals = values.reshape(_N_ROWS, _UNITS)
    return pl.pallas_call(
        _tc_body,
        grid=(_TC_ROWS // _ROWS_PER_BLOCK,),
        in_specs=[pl.BlockSpec((_ROWS_PER_BLOCK, _UNITS), lambda i: (i, 0))],
        out_specs=pl.BlockSpec((_ROWS_PER_BLOCK, _UNITS), lambda i: (i, 0)),
        out_shape=jax.ShapeDtypeStruct((_N_ROWS, _UNITS), jnp.float32),
        compiler_params=pltpu.CompilerParams(
            dimension_semantics=("parallel",)),
    )(vals)


def _sc_kernel_body(v_hbm, out_hbm, vin0, vin1, vout0, vout1,
                    si0, si1, so0, so1):
    wid = lax.axis_index("s") * 2 + lax.axis_index("c")
    wbase = wid * _PER_W

    def in_cp(c, buf, sem):
        off = _SC_BASE + wbase + c * _CHUNK
        return pltpu.make_async_copy(v_hbm.at[pl.ds(off, _CHUNK)], buf, sem)

    def out_cp(c, buf, sem):
        off = wbase + c * _CHUNK
        return pltpu.make_async_copy(buf, out_hbm.at[pl.ds(off, _CHUNK)], sem)

    def compute(vin, vout, c):
        base_ctr = _SC_BASE + wbase + c * _CHUNK + _KS[1]

        def vec_body(j, _):
            s = base_ctr + j * (16 * _UNROLL)
            iota = lax.iota(jnp.int32, 16)
            for u in range(_UNROLL):
                ctr = iota + (s + u * 16)
                bits = _threefry_bits(ctr)
                sl = pl.ds(j * (16 * _UNROLL) + u * 16, 16)
                vout[sl] = jnp.where(bits >= 0, vin[sl] * 2.0, 0.0)
            return 0

        lax.fori_loop(0, _CHUNK // (16 * _UNROLL), vec_body, 0)

    in_cp(0, vin0, si0).start()

    def pair_body(it, _):
        c0 = 2 * it
        c1 = c0 + 1
        in_cp(c1, vin1, si1).start()
        in_cp(c0, vin0, si0).wait()

        @pl.when(it > 0)
        def _():
            out_cp(c0 - 2, vout0, so0).wait()

        compute(vin0, vout0, c0)
        out_cp(c0, vout0, so0).start()

        @pl.when(it + 1 < _N_PAIR)
        def _():
            in_cp(c0 + 2, vin0, si0).start()

        in_cp(c1, vin1, si1).wait()

        @pl.when(it > 0)
        def _():
            out_cp(c1 - 2, vout1, so1).wait()

        compute(vin1, vout1, c1)
        out_cp(c1, vout1, so1).start()
        return 0

    lax.fori_loop(0, _N_PAIR, pair_body, 0)
    out_cp(_N_CHUNK - 2, vout0, so0).wait()
    out_cp(_N_CHUNK - 1, vout1, so1).wait()


def _sc_call(values):
    mesh = plsc.VectorSubcoreMesh(core_axis_name="c", subcore_axis_name="s")
    k = functools.partial(
        pl.kernel,
        mesh=mesh,
        out_type=jax.ShapeDtypeStruct((_SC_ELEMS,), jnp.float32),
        scratch_types=[
            pltpu.VMEM((_CHUNK,), jnp.float32),
            pltpu.VMEM((_CHUNK,), jnp.float32),
            pltpu.VMEM((_CHUNK,), jnp.float32),
            pltpu.VMEM((_CHUNK,), jnp.float32),
            pltpu.SemaphoreType.DMA,
            pltpu.SemaphoreType.DMA,
            pltpu.SemaphoreType.DMA,
            pltpu.SemaphoreType.DMA,
        ],
    )(_sc_kernel_body)
    return k(values)


def kernel(values, indices):
    del indices
    tc_out = _tc_call(values)
    sc_out = _sc_call(values)
    return lax.dynamic_update_slice(tc_out.reshape(-1), sc_out, (_SC_BASE,))


# speedup vs baseline: 1.1998x; 1.0088x over previous
import functools

import jax
import jax.numpy as jnp
from jax import lax
from jax.experimental import pallas as pl
from jax.experimental.pallas import tpu as pltpu
from jax.experimental.pallas import tpu_sc as plsc

_UNITS = 128
_N_ROWS = 160000
_TOTAL = _N_ROWS * _UNITS

_SC_ROWS = 41600
_TC_ROWS = _N_ROWS - _SC_ROWS
_SC_BASE = _TC_ROWS * _UNITS
_SC_ELEMS = _SC_ROWS * _UNITS

_ROWS_PER_BLOCK = 1600

_NW = 32
_PER_W = _SC_ELEMS // _NW
_CHUNK = 20800
_N_CHUNK = _PER_W // _CHUNK
_N_PAIR = _N_CHUNK // 2
_UNROLL = 4

_KS = (0, 42, 0x1BD11BDA ^ 42)
_ROT = ((13, 15, 26, 6), (17, 29, 16, 24))
_INJ = ((1, 2, 0), (2, 0, 1), (0, 1, 0), (1, 2, 1), (2, 0, 0))


def _rotl(x, r):
    return (x << r) | lax.shift_right_logical(x, 32 - r)


def _rounds(x0, x1, rots):
    for r in rots:
        x0 = x0 + x1
        x1 = _rotl(x1, r) ^ x0
    return x0, x1


def _threefry_bits(ctr):
    x0 = ctr
    x1 = _rotl(ctr, _ROT[0][0]) ^ ctr
    x0, x1 = _rounds(x0, x1, _ROT[0][1:])
    for i, (a, b, grp) in enumerate(_INJ, start=1):
        x0 = x0 + jnp.int32(_KS[a])
        x1 = x1 + jnp.int32(_KS[b] + i)
        if i < 5:
            x0, x1 = _rounds(x0, x1, _ROT[(grp + 1) % 2])
    return x0 ^ x1


def _tc_body(v_ref, o_ref):
    pid = pl.program_id(0)
    shape = v_ref.shape
    base = pid * (shape[0] * shape[1])
    row = lax.broadcasted_iota(jnp.int32, shape, 0)
    col = lax.broadcasted_iota(jnp.int32, shape, 1)
    ctr = (base + _KS[1]) + ((row << 7) | col)
    bits = _threefry_bits(ctr)
    o_ref[...] = jnp.where(bits >= 0, v_ref[...] * 2.0, 0.0)


def _tc_call(values):
    vals = values.reshape(_N_ROWS, _UNITS)
    return pl.pallas_call(
        _tc_body,
        grid=(_TC_ROWS // _ROWS_PER_BLOCK,),
        in_specs=[pl.BlockSpec((_ROWS_PER_BLOCK, _UNITS), lambda i: (i, 0))],
        out_specs=pl.BlockSpec((_ROWS_PER_BLOCK, _UNITS), lambda i: (i, 0)),
        out_shape=jax.ShapeDtypeStruct((_N_ROWS, _UNITS), jnp.float32),
        compiler_params=pltpu.CompilerParams(
            dimension_semantics=("parallel",)),
    )(vals)


def _sc_kernel_body(v_hbm, out_hbm, vin0, vin1, vout0, vout1,
                    si0, si1, so0, so1):
    wid = lax.axis_index("s") * 2 + lax.axis_index("c")
    wbase = wid * _PER_W

    def in_cp(c, buf, sem):
        off = _SC_BASE + wbase + c * _CHUNK
        return pltpu.make_async_copy(v_hbm.at[pl.ds(off, _CHUNK)], buf, sem)

    def out_cp(c, buf, sem):
        off = wbase + c * _CHUNK
        return pltpu.make_async_copy(buf, out_hbm.at[pl.ds(off, _CHUNK)], sem)

    def compute(vin, vout, c):
        base_ctr = _SC_BASE + wbase + c * _CHUNK + _KS[1]

        def vec_body(j, _):
            s = base_ctr + j * (16 * _UNROLL)
            iota = lax.iota(jnp.int32, 16)
            for u in range(_UNROLL):
                ctr = iota + (s + u * 16)
                bits = _threefry_bits(ctr)
                sl = pl.ds(j * (16 * _UNROLL) + u * 16, 16)
                vout[sl] = jnp.where(bits >= 0, vin[sl] * 2.0, 0.0)
            return 0

        lax.fori_loop(0, _CHUNK // (16 * _UNROLL), vec_body, 0)

    in_cp(0, vin0, si0).start()

    def pair_body(it, _):
        c0 = 2 * it
        c1 = c0 + 1
        in_cp(c1, vin1, si1).start()
        in_cp(c0, vin0, si0).wait()

        @pl.when(it > 0)
        def _():
            out_cp(c0 - 2, vout0, so0).wait()

        compute(vin0, vout0, c0)
        out_cp(c0, vout0, so0).start()

        @pl.when(it + 1 < _N_PAIR)
        def _():
            in_cp(c0 + 2, vin0, si0).start()

        in_cp(c1, vin1, si1).wait()

        @pl.when(it > 0)
        def _():
            out_cp(c1 - 2, vout1, so1).wait()

        compute(vin1, vout1, c1)
        out_cp(c1, vout1, so1).start()
        return 0

    lax.fori_loop(0, _N_PAIR, pair_body, 0)
    out_cp(_N_CHUNK - 2, vout0, so0).wait()
    out_cp(_N_CHUNK - 1, vout1, so1).wait()


def _sc_call(values):
    mesh = plsc.VectorSubcoreMesh(core_axis_name="c", subcore_axis_name="s")
    k = functools.partial(
        pl.kernel,
        mesh=mesh,
        out_type=jax.ShapeDtypeStruct((_SC_ELEMS,), jnp.float32),
        scratch_types=[
            pltpu.VMEM((_CHUNK,), jnp.float32),
            pltpu.VMEM((_CHUNK,), jnp.float32),
            pltpu.VMEM((_CHUNK,), jnp.float32),
            pltpu.VMEM((_CHUNK,), jnp.float32),
            pltpu.SemaphoreType.DMA,
            pltpu.SemaphoreType.DMA,
            pltpu.SemaphoreType.DMA,
            pltpu.SemaphoreType.DMA,
        ],
    )(_sc_kernel_body)
    return k(values)


def kernel(values, indices):
    del indices
    tc_out = _tc_call(values)
    sc_out = _sc_call(values)
    return lax.dynamic_update_slice(tc_out.reshape(-1), sc_out, (_SC_BASE,))


# speedup vs baseline: 1.2004x; 1.0005x over previous
import functools

import jax
import jax.numpy as jnp
from jax import lax
from jax.experimental import pallas as pl
from jax.experimental.pallas import tpu as pltpu
from jax.experimental.pallas import tpu_sc as plsc

_UNITS = 128
_N_ROWS = 160000
_TOTAL = _N_ROWS * _UNITS

_SC_ROWS = 41600
_TC_ROWS = _N_ROWS - _SC_ROWS
_SC_BASE = _TC_ROWS * _UNITS
_SC_ELEMS = _SC_ROWS * _UNITS

_ROWS_PER_BLOCK = 3200

_NW = 32
_PER_W = _SC_ELEMS // _NW
_CHUNK = 20800
_N_CHUNK = _PER_W // _CHUNK
_N_PAIR = _N_CHUNK // 2
_UNROLL = 4

_KS = (0, 42, 0x1BD11BDA ^ 42)
_ROT = ((13, 15, 26, 6), (17, 29, 16, 24))
_INJ = ((1, 2, 0), (2, 0, 1), (0, 1, 0), (1, 2, 1), (2, 0, 0))


def _rotl(x, r):
    return (x << r) | lax.shift_right_logical(x, 32 - r)


def _rounds(x0, x1, rots):
    for r in rots:
        x0 = x0 + x1
        x1 = _rotl(x1, r) ^ x0
    return x0, x1


def _threefry_bits(ctr):
    x0 = ctr
    x1 = _rotl(ctr, _ROT[0][0]) ^ ctr
    x0, x1 = _rounds(x0, x1, _ROT[0][1:])
    for i, (a, b, grp) in enumerate(_INJ, start=1):
        x0 = x0 + jnp.int32(_KS[a])
        x1 = x1 + jnp.int32(_KS[b] + i)
        if i < 5:
            x0, x1 = _rounds(x0, x1, _ROT[(grp + 1) % 2])
    return x0 ^ x1


def _tc_body(v_ref, o_ref):
    pid = pl.program_id(0)
    shape = v_ref.shape
    base = pid * (shape[0] * shape[1])
    row = lax.broadcasted_iota(jnp.int32, shape, 0)
    col = lax.broadcasted_iota(jnp.int32, shape, 1)
    ctr = (base + _KS[1]) + ((row << 7) | col)
    bits = _threefry_bits(ctr)
    o_ref[...] = jnp.where(bits >= 0, v_ref[...] * 2.0, 0.0)


def _tc_call(values):
    vals = values.reshape(_N_ROWS, _UNITS)
    return pl.pallas_call(
        _tc_body,
        grid=(_TC_ROWS // _ROWS_PER_BLOCK,),
        in_specs=[pl.BlockSpec((_ROWS_PER_BLOCK, _UNITS), lambda i: (i, 0))],
        out_specs=pl.BlockSpec((_ROWS_PER_BLOCK, _UNITS), lambda i: (i, 0)),
        out_shape=jax.ShapeDtypeStruct((_N_ROWS, _UNITS), jnp.float32),
        compiler_params=pltpu.CompilerParams(
            dimension_semantics=("parallel",)),
    )(vals)


def _sc_kernel_body(v_hbm, out_hbm, vin0, vin1, vout0, vout1,
                    si0, si1, so0, so1):
    wid = lax.axis_index("s") * 2 + lax.axis_index("c")
    wbase = wid * _PER_W

    def in_cp(c, buf, sem):
        off = _SC_BASE + wbase + c * _CHUNK
        return pltpu.make_async_copy(v_hbm.at[pl.ds(off, _CHUNK)], buf, sem)

    def out_cp(c, buf, sem):
        off = wbase + c * _CHUNK
        return pltpu.make_async_copy(buf, out_hbm.at[pl.ds(off, _CHUNK)], sem)

    def compute(vin, vout, c):
        base_ctr = _SC_BASE + wbase + c * _CHUNK + _KS[1]

        def vec_body(j, _):
            s = base_ctr + j * (16 * _UNROLL)
            iota = lax.iota(jnp.int32, 16)
            for u in range(_UNROLL):
                ctr = iota + (s + u * 16)
                bits = _threefry_bits(ctr)
                sl = pl.ds(j * (16 * _UNROLL) + u * 16, 16)
                vout[sl] = jnp.where(bits >= 0, vin[sl] * 2.0, 0.0)
            return 0

        lax.fori_loop(0, _CHUNK // (16 * _UNROLL), vec_body, 0)

    in_cp(0, vin0, si0).start()

    def pair_body(it, _):
        c0 = 2 * it
        c1 = c0 + 1
        in_cp(c1, vin1, si1).start()
        in_cp(c0, vin0, si0).wait()

        @pl.when(it > 0)
        def _():
            out_cp(c0 - 2, vout0, so0).wait()

        compute(vin0, vout0, c0)
        out_cp(c0, vout0, so0).start()

        @pl.when(it + 1 < _N_PAIR)
        def _():
            in_cp(c0 + 2, vin0, si0).start()

        in_cp(c1, vin1, si1).wait()

        @pl.when(it > 0)
        def _():
            out_cp(c1 - 2, vout1, so1).wait()

        compute(vin1, vout1, c1)
        out_cp(c1, vout1, so1).start()
        return 0

    lax.fori_loop(0, _N_PAIR, pair_body, 0)
    out_cp(_N_CHUNK - 2, vout0, so0).wait()
    out_cp(_N_CHUNK - 1, vout1, so1).wait()


def _sc_call(values):
    mesh = plsc.VectorSubcoreMesh(core_axis_name="c", subcore_axis_name="s")
    k = functools.partial(
        pl.kernel,
        mesh=mesh,
        out_type=jax.ShapeDtypeStruct((_SC_ELEMS,), jnp.float32),
        scratch_types=[
            pltpu.VMEM((_CHUNK,), jnp.float32),
            pltpu.VMEM((_CHUNK,), jnp.float32),
            pltpu.VMEM((_CHUNK,), jnp.float32),
            pltpu.VMEM((_CHUNK,), jnp.float32),
            pltpu.SemaphoreType.DMA,
            pltpu.SemaphoreType.DMA,
            pltpu.SemaphoreType.DMA,
            pltpu.SemaphoreType.DMA,
        ],
    )(_sc_kernel_body)
    return k(values)


def kernel(values, indices):
    del indices
    tc_out = _tc_call(values)
    sc_out = _sc_call(values)
    return lax.dynamic_update_slice(tc_out.reshape(-1), sc_out, (_SC_BASE,))


# speedup vs baseline: 1.2034x; 1.0026x over previous
import functools

import jax
import jax.numpy as jnp
from jax import lax
from jax.experimental import pallas as pl
from jax.experimental.pallas import tpu as pltpu
from jax.experimental.pallas import tpu_sc as plsc

_UNITS = 128
_N_ROWS = 160000
_TOTAL = _N_ROWS * _UNITS

_SC_ROWS = 41600
_TC_ROWS = _N_ROWS - _SC_ROWS
_SC_BASE = _TC_ROWS * _UNITS
_SC_ELEMS = _SC_ROWS * _UNITS

_ROWS_PER_BLOCK = 3200

_NW = 32
_PER_W = _SC_ELEMS // _NW
_CHUNK = 20800
_N_CHUNK = _PER_W // _CHUNK
_N_PAIR = _N_CHUNK // 2
_UNROLL = 4

_KS = (0, 42, 0x1BD11BDA ^ 42)
_ROT = ((13, 15, 26, 6), (17, 29, 16, 24))
_INJ = ((1, 2, 0), (2, 0, 1), (0, 1, 0), (1, 2, 1), (2, 0, 0))


def _rotl(x, r):
    return (x << r) | lax.shift_right_logical(x, 32 - r)


def _rounds(x0, x1, rots):
    for r in rots:
        x0 = x0 + x1
        x1 = _rotl(x1, r) ^ x0
    return x0, x1


def _threefry_bits(ctr):
    x0 = ctr
    x1 = _rotl(ctr, _ROT[0][0]) ^ ctr
    x0, x1 = _rounds(x0, x1, _ROT[0][1:])
    for i, (a, b, grp) in enumerate(_INJ, start=1):
        x0 = x0 + jnp.int32(_KS[a])
        x1 = x1 + jnp.int32(_KS[b] + i)
        if i < 5:
            x0, x1 = _rounds(x0, x1, _ROT[(grp + 1) % 2])
    return x0 ^ x1


def _tc_body(v_ref, o_ref):
    pid = pl.program_id(0)
    shape = v_ref.shape
    base = pid * (shape[0] * shape[1])
    row = lax.broadcasted_iota(jnp.int32, shape, 0)
    col = lax.broadcasted_iota(jnp.int32, shape, 1)
    ctr = (base + _KS[1]) + ((row << 7) | col)
    bits = _threefry_bits(ctr)
    o_ref[...] = jnp.where(bits >= 0, v_ref[...] * 2.0, 0.0)


def _tc_call(values):
    vals = values.reshape(_N_ROWS, _UNITS)
    return pl.pallas_call(
        _tc_body,
        grid=(_TC_ROWS // _ROWS_PER_BLOCK,),
        in_specs=[pl.BlockSpec((_ROWS_PER_BLOCK, _UNITS), lambda i: (i, 0))],
        out_specs=pl.BlockSpec((_ROWS_PER_BLOCK, _UNITS), lambda i: (i, 0)),
        out_shape=jax.ShapeDtypeStruct((_N_ROWS, _UNITS), jnp.float32),
        compiler_params=pltpu.CompilerParams(
            dimension_semantics=("parallel",)),
    )(vals)


def _sc_kernel_body(v_hbm, out_hbm, vin0, vin1, vout0, vout1,
                    si0, si1, so0, so1):
    wid = lax.axis_index("s") * 2 + lax.axis_index("c")
    wbase = wid * _PER_W

    def in_cp(c, buf, sem):
        off = _SC_BASE + wbase + c * _CHUNK
        return pltpu.make_async_copy(v_hbm.at[pl.ds(off, _CHUNK)], buf, sem)

    def out_cp(c, buf, sem):
        off = wbase + c * _CHUNK
        return pltpu.make_async_copy(buf, out_hbm.at[pl.ds(off, _CHUNK)], sem)

    def compute(vin, vout, c):
        base_ctr = _SC_BASE + wbase + c * _CHUNK + _KS[1]

        @plsc.parallel_loop(0, _CHUNK, 16 * _UNROLL)
        def vec_body(s_loc):
            iota = lax.iota(jnp.int32, 16)
            for u in range(_UNROLL):
                ctr = iota + (base_ctr + s_loc + u * 16)
                bits = _threefry_bits(ctr)
                sl = pl.ds(s_loc + u * 16, 16)
                vout[sl] = jnp.where(bits >= 0, vin[sl] * 2.0, 0.0)

    in_cp(0, vin0, si0).start()

    def pair_body(it, _):
        c0 = 2 * it
        c1 = c0 + 1
        in_cp(c1, vin1, si1).start()
        in_cp(c0, vin0, si0).wait()

        @pl.when(it > 0)
        def _():
            out_cp(c0 - 2, vout0, so0).wait()

        compute(vin0, vout0, c0)
        out_cp(c0, vout0, so0).start()

        @pl.when(it + 1 < _N_PAIR)
        def _():
            in_cp(c0 + 2, vin0, si0).start()

        in_cp(c1, vin1, si1).wait()

        @pl.when(it > 0)
        def _():
            out_cp(c1 - 2, vout1, so1).wait()

        compute(vin1, vout1, c1)
        out_cp(c1, vout1, so1).start()
        return 0

    lax.fori_loop(0, _N_PAIR, pair_body, 0)
    out_cp(_N_CHUNK - 2, vout0, so0).wait()
    out_cp(_N_CHUNK - 1, vout1, so1).wait()


def _sc_call(values):
    mesh = plsc.VectorSubcoreMesh(core_axis_name="c", subcore_axis_name="s")
    k = functools.partial(
        pl.kernel,
        mesh=mesh,
        out_type=jax.ShapeDtypeStruct((_SC_ELEMS,), jnp.float32),
        scratch_types=[
            pltpu.VMEM((_CHUNK,), jnp.float32),
            pltpu.VMEM((_CHUNK,), jnp.float32),
            pltpu.VMEM((_CHUNK,), jnp.float32),
            pltpu.VMEM((_CHUNK,), jnp.float32),
            pltpu.SemaphoreType.DMA,
            pltpu.SemaphoreType.DMA,
            pltpu.SemaphoreType.DMA,
            pltpu.SemaphoreType.DMA,
        ],
    )(_sc_kernel_body)
    return k(values)


def kernel(values, indices):
    del indices
    tc_out = _tc_call(values)
    sc_out = _sc_call(values)
    return lax.dynamic_update_slice(tc_out.reshape(-1), sc_out, (_SC_BASE,))


# speedup vs baseline: 1.2235x; 1.0167x over previous
import functools

import jax
import jax.numpy as jnp
from jax import lax
from jax.experimental import pallas as pl
from jax.experimental.pallas import tpu as pltpu
from jax.experimental.pallas import tpu_sc as plsc

_UNITS = 128
_N_ROWS = 160000
_TOTAL = _N_ROWS * _UNITS

_SC_ROWS = 40960
_TC_ROWS = _N_ROWS - _SC_ROWS
_SC_BASE = _TC_ROWS * _UNITS
_SC_ELEMS = _SC_ROWS * _UNITS

_ROWS_PER_BLOCK = 1920

_NW = 32
_PER_W = _SC_ELEMS // _NW
_CHUNK = 20480
_N_CHUNK = _PER_W // _CHUNK
_N_PAIR = _N_CHUNK // 2
_UNROLL = 4

_KS = (0, 42, 0x1BD11BDA ^ 42)
_ROT = ((13, 15, 26, 6), (17, 29, 16, 24))
_INJ = ((1, 2, 0), (2, 0, 1), (0, 1, 0), (1, 2, 1), (2, 0, 0))


def _rotl(x, r):
    return (x << r) | lax.shift_right_logical(x, 32 - r)


def _rounds(x0, x1, rots):
    for r in rots:
        x0 = x0 + x1
        x1 = _rotl(x1, r) ^ x0
    return x0, x1


def _threefry_bits(ctr):
    x0 = ctr
    x1 = _rotl(ctr, _ROT[0][0]) ^ ctr
    x0, x1 = _rounds(x0, x1, _ROT[0][1:])
    for i, (a, b, grp) in enumerate(_INJ, start=1):
        x0 = x0 + jnp.int32(_KS[a])
        x1 = x1 + jnp.int32(_KS[b] + i)
        if i < 5:
            x0, x1 = _rounds(x0, x1, _ROT[(grp + 1) % 2])
    return x0 ^ x1


def _tc_body(v_ref, o_ref, pat_ref):
    pid = pl.program_id(0)
    shape = v_ref.shape

    @pl.when(pid == 0)
    def _():
        row = lax.broadcasted_iota(jnp.int32, shape, 0)
        col = lax.broadcasted_iota(jnp.int32, shape, 1)
        pat_ref[...] = (row << 7) | col

    base = pid * (shape[0] * shape[1])
    ctr = (base + _KS[1]) + pat_ref[...]
    bits = _threefry_bits(ctr)
    o_ref[...] = jnp.where(bits >= 0, v_ref[...] * 2.0, 0.0)


def _tc_call(values):
    vals = values.reshape(_N_ROWS, _UNITS)
    return pl.pallas_call(
        _tc_body,
        grid=(_TC_ROWS // _ROWS_PER_BLOCK,),
        in_specs=[pl.BlockSpec((_ROWS_PER_BLOCK, _UNITS), lambda i: (i, 0))],
        out_specs=pl.BlockSpec((_ROWS_PER_BLOCK, _UNITS), lambda i: (i, 0)),
        out_shape=jax.ShapeDtypeStruct((_N_ROWS, _UNITS), jnp.float32),
        scratch_shapes=[pltpu.VMEM((_ROWS_PER_BLOCK, _UNITS), jnp.int32)],
        compiler_params=pltpu.CompilerParams(
            dimension_semantics=("arbitrary",)),
    )(vals)


def _sc_kernel_body(v_hbm, out_hbm, vin0, vin1, vout0, vout1,
                    si0, si1, so0, so1):
    wid = lax.axis_index("s") * 2 + lax.axis_index("c")
    wbase = wid * _PER_W

    def in_cp(c, buf, sem):
        off = _SC_BASE + wbase + c * _CHUNK
        return pltpu.make_async_copy(v_hbm.at[pl.ds(off, _CHUNK)], buf, sem)

    def out_cp(c, buf, sem):
        off = wbase + c * _CHUNK
        return pltpu.make_async_copy(buf, out_hbm.at[pl.ds(off, _CHUNK)], sem)

    def compute(vin, vout, c):
        base_ctr = _SC_BASE + wbase + c * _CHUNK + _KS[1]

        @plsc.parallel_loop(0, _CHUNK, 16 * _UNROLL)
        def vec_body(s_loc):
            iota = lax.iota(jnp.int32, 16)
            for u in range(_UNROLL):
                ctr = iota + (base_ctr + s_loc + u * 16)
                bits = _threefry_bits(ctr)
                sl = pl.ds(s_loc + u * 16, 16)
                vout[sl] = jnp.where(bits >= 0, vin[sl] * 2.0, 0.0)

    in_cp(0, vin0, si0).start()

    def pair_body(it, _):
        c0 = 2 * it
        c1 = c0 + 1
        in_cp(c1, vin1, si1).start()
        in_cp(c0, vin0, si0).wait()

        @pl.when(it > 0)
        def _():
            out_cp(c0 - 2, vout0, so0).wait()

        compute(vin0, vout0, c0)
        out_cp(c0, vout0, so0).start()

        @pl.when(it + 1 < _N_PAIR)
        def _():
            in_cp(c0 + 2, vin0, si0).start()

        in_cp(c1, vin1, si1).wait()

        @pl.when(it > 0)
        def _():
            out_cp(c1 - 2, vout1, so1).wait()

        compute(vin1, vout1, c1)
        out_cp(c1, vout1, so1).start()
        return 0

    lax.fori_loop(0, _N_PAIR, pair_body, 0)
    out_cp(_N_CHUNK - 2, vout0, so0).wait()
    out_cp(_N_CHUNK - 1, vout1, so1).wait()


def _sc_call(values):
    mesh = plsc.VectorSubcoreMesh(core_axis_name="c", subcore_axis_name="s")
    k = functools.partial(
        pl.kernel,
        mesh=mesh,
        out_type=jax.ShapeDtypeStruct((_SC_ELEMS,), jnp.float32),
        scratch_types=[
            pltpu.VMEM((_CHUNK,), jnp.float32),
            pltpu.VMEM((_CHUNK,), jnp.float32),
            pltpu.VMEM((_CHUNK,), jnp.float32),
            pltpu.VMEM((_CHUNK,), jnp.float32),
            pltpu.SemaphoreType.DMA,
            pltpu.SemaphoreType.DMA,
            pltpu.SemaphoreType.DMA,
            pltpu.SemaphoreType.DMA,
        ],
    )(_sc_kernel_body)
    return k(values)


def kernel(values, indices):
    del indices
    tc_out = _tc_call(values)
    sc_out = _sc_call(values)
    return lax.dynamic_update_slice(tc_out.reshape(-1), sc_out, (_SC_BASE,))
